# baseline (speedup 1.0000x reference)
import jax
import jax.numpy as jnp
from jax import lax
from jax.experimental import pallas as pl
from jax.experimental.pallas import tpu as pltpu
from jax.experimental.pallas import tpu_sc as plsc

_VOCAB = 1000
_W = 26
_ROW = _W * _VOCAB
_NC = 2
_NS = 16
_NW = _NC * _NS
_RPC = 1
_BATCH = 4096
_ROWS_PER_TILE = _BATCH // _NW
_CHUNKS = _ROWS_PER_TILE // _RPC
_ENT = _RPC * _W
_ENT_PAD = 32
_CHUNK_F32 = _RPC * _ROW
_NBUF = 3


def _sc_body(pos_hbm, out_hbm, pos_vmem, vbuf, sem0, sem1, sem2):
    cid = lax.axis_index("c")
    sid = lax.axis_index("s")
    wid = sid * _NC + cid
    row_base = wid * _ROWS_PER_TILE
    sems = (sem0, sem1, sem2)

    pltpu.sync_copy(pos_hbm.at[wid], pos_vmem)

    zeros16 = jnp.zeros((16,), jnp.float32)

    def _zero_row(rr, carry):
        r0 = rr // _W
        r1 = rr - _W * r0

        def _zero_col(k, c2):
            vbuf[r0, r1, pl.ds(k * 16, 16)] = zeros16
            return c2

        lax.fori_loop(0, _VOCAB // 16, _zero_col, None)
        tail = jnp.full((16,), _VOCAB - 16, jnp.int32) + lax.iota(jnp.int32, 16)
        plsc.store_scatter(
            vbuf,
            [jnp.full((16,), r0, jnp.int32), jnp.full((16,), r1, jnp.int32), tail],
            zeros16,
        )
        return carry

    lax.fori_loop(0, _NBUF * _RPC * _W, _zero_row, None)

    ones16 = jnp.ones((16,), jnp.float32)

    def _scatter_chunk(c, b, vals):
        for g in range(_ENT_PAD // 16):
            e = lax.iota(jnp.int32, 16) + (16 * g)
            r = e // _W
            i0 = r + _RPC * b
            i1 = e - _W * r
            i2 = pos_vmem[c, pl.ds(16 * g, 16)]
            plsc.store_scatter(vbuf, [i0, i1, i2], vals, mask=e < _ENT)

    def _chunk_group(t, carry):
        for b in range(_NBUF):
            c = _NBUF * t + b
            row0 = row_base + c * _RPC

            @pl.when(t >= 1)
            def _wait_and_clear(b=b, c=c, row0=row0):
                pltpu.make_async_copy(
                    vbuf.at[pl.ds(_RPC * b, _RPC)],
                    out_hbm.at[pl.ds(row0 - _NBUF * _RPC, _RPC)],
                    sems[b],
                ).wait()
                _scatter_chunk(c - _NBUF, b, zeros16)

            _scatter_chunk(c, b, ones16)
            pltpu.make_async_copy(
                vbuf.at[pl.ds(_RPC * b, _RPC)],
                out_hbm.at[pl.ds(row0, _RPC)],
                sems[b],
            ).start()
        return carry

    n_groups = _CHUNKS // _NBUF
    lax.fori_loop(0, n_groups, _chunk_group, None)
    for b in range(_CHUNKS - n_groups * _NBUF):
        c = n_groups * _NBUF + b
        row0 = row_base + c * _RPC
        pltpu.make_async_copy(
            vbuf.at[pl.ds(_RPC * b, _RPC)],
            out_hbm.at[pl.ds(row0 - _NBUF * _RPC, _RPC)],
            sems[b],
        ).wait()
        _scatter_chunk(c - _NBUF, b, zeros16)
        _scatter_chunk(c, b, ones16)
        pltpu.make_async_copy(
            vbuf.at[pl.ds(_RPC * b, _RPC)],
            out_hbm.at[pl.ds(row0, _RPC)],
            sems[b],
        ).start()

    for b in range(_NBUF):
        pltpu.make_async_copy(
            vbuf.at[pl.ds(_RPC * b, _RPC)],
            out_hbm.at[pl.ds(row_base, _RPC)],
            sems[b],
        ).wait()


def kernel(inputs):
    b, w = inputs.shape
    idx32 = inputs.astype(jnp.int32)
    pos = idx32.reshape(_NW, _CHUNKS, _ENT)
    pos = jnp.pad(pos, ((0, 0), (0, 0), (0, _ENT_PAD - _ENT)))

    mesh = plsc.VectorSubcoreMesh(core_axis_name="c", subcore_axis_name="s")
    fn = pl.kernel(
        _sc_body,
        out_type=jax.ShapeDtypeStruct((b, w, _VOCAB), jnp.float32),
        mesh=mesh,
        compiler_params=pltpu.CompilerParams(needs_layout_passes=False),
        scratch_types=[
            pltpu.VMEM((_CHUNKS, _ENT_PAD), jnp.int32),
            pltpu.VMEM((_NBUF * _RPC, _W, _VOCAB), jnp.float32),
            pltpu.SemaphoreType.DMA,
            pltpu.SemaphoreType.DMA,
            pltpu.SemaphoreType.DMA,
        ],
    )
    return fn(pos)


# speedup vs baseline: 1.0296x; 1.0296x over previous
import jax
import jax.numpy as jnp
from jax.experimental import pallas as pl
from jax.experimental.pallas import tpu as pltpu

_VOCAB = 1000
_B = 16
_K = 6
_WSPLIT = 16


def _onehot_block(idx_ref, out_ref, vbuf, sems):
    i = pl.program_id(0)
    n = pl.num_programs(0)
    slot = jax.lax.rem(i, _K)
    w = vbuf.shape[2]

    def _copies(j, base):
        return (
            pltpu.make_async_copy(
                vbuf.at[j, :, pl.ds(0, _WSPLIT)],
                out_ref.at[pl.ds(base, _B), pl.ds(0, _WSPLIT)],
                sems.at[j, 0],
            ),
            pltpu.make_async_copy(
                vbuf.at[j, :, pl.ds(_WSPLIT, w - _WSPLIT)],
                out_ref.at[pl.ds(base, _B), pl.ds(_WSPLIT, w - _WSPLIT)],
                sems.at[j, 1],
            ),
        )

    for j in range(_K):
        @pl.when(jnp.logical_and(slot == j, i >= _K))
        def _wait_prev(j=j):
            for c in _copies(j, (i - _K) * _B):
                c.wait()

    idx = idx_ref[...]
    iota = jax.lax.broadcasted_iota(jnp.int32, vbuf.shape[1:], 2)
    block = (iota == idx[:, :, None]).astype(jnp.float32)

    for j in range(_K):
        @pl.when(slot == j)
        def _start(j=j):
            vbuf[j] = block
            for c in _copies(j, i * _B):
                c.start()

    @pl.when(i == n - 1)
    def _drain():
        for j in range(_K):
            for c in _copies(j, 0):
                c.wait()


def kernel(inputs):
    b, w = inputs.shape
    idx = inputs.astype(jnp.int32)
    grid = b // _B
    return pl.pallas_call(
        _onehot_block,
        grid=(grid,),
        in_specs=[pl.BlockSpec((_B, w), lambda i: (i, 0))],
        out_specs=pl.BlockSpec(memory_space=pl.ANY),
        out_shape=jax.ShapeDtypeStruct((b, w, _VOCAB), jnp.float32),
        scratch_shapes=[
            pltpu.VMEM((_K, _B, w, _VOCAB), jnp.float32),
            pltpu.SemaphoreType.DMA((_K, 2)),
        ],
    )(idx)
